# baseline (speedup 1.0000x reference)
import functools

import jax
import jax.numpy as jnp
from jax.experimental import pallas as pl
from jax.experimental.pallas import tpu as pltpu

_IN = 16
_H2 = 64
_OUT = 16


def _fused_kernel(x0_ref, x1_ref, w1_ref, b1_ref, w2m_ref, b2m_ref,
                  w2a_ref, b2a_ref, aug_ref, ml_ref):
    half = x0_ref.shape[0]
    for k, x_ref in enumerate((x0_ref, x1_ref)):
        rows = pl.ds(k * half, half)
        x = x_ref[...]
        h = jnp.maximum(
            jnp.dot(x, w1_ref[...], preferred_element_type=jnp.float32)
            + b1_ref[...], 0.0)

        ml_ref[rows, :] = (jnp.dot(h, w2m_ref[...],
                                   preferred_element_type=jnp.float32)
                           + b2m_ref[...])

        a = (jnp.dot(h, w2a_ref[...], preferred_element_type=jnp.float32)
             + b2a_ref[...])
        m = jnp.max(a, axis=1, keepdims=True)
        s = a - m
        lse = jnp.log(jnp.sum(jnp.exp(s), axis=1, keepdims=True))
        aug_ref[rows, :] = s - lse


@jax.jit
def _forward(x, w1, b1, w2, b2):
    x = x.astype(jnp.float32)
    B = x.shape[0]

    w1u = jax.lax.slice(w1, (0, 0), (_IN, _H2))
    b1u = jax.lax.slice(b1, (0, 0), (1, _H2))
    w2m = jax.lax.slice(w2, (0, 0), (_H2, _OUT))
    b2m = jax.lax.slice(b2, (0, 0), (1, _OUT))
    w2a = jax.lax.slice(w2, (0, _OUT), (_H2, 2 * _OUT))
    b2a = jax.lax.slice(b2, (0, _OUT), (1, 2 * _OUT))

    TB = 16384
    num_tiles = pl.cdiv(B, TB)
    Bp = num_tiles * TB
    if Bp != B:
        x = jnp.pad(x, ((0, Bp - B), (0, 0)))

    flops = 2 * Bp * (_IN * _H2 + _H2 * 2 * _OUT)
    bytes_accessed = 4 * (Bp * (_IN + 2 * _OUT)
                          + _IN * _H2 + _H2 * 2 * _OUT + _H2 + 2 * _OUT)

    aug, ml = pl.pallas_call(
        _fused_kernel,
        out_shape=(jax.ShapeDtypeStruct((Bp, _OUT), jnp.float32),
                   jax.ShapeDtypeStruct((Bp, _OUT), jnp.float32)),
        grid=(num_tiles,),
        in_specs=[
            pl.BlockSpec((TB // 2, _IN), lambda i: (2 * i, 0)),
            pl.BlockSpec((TB // 2, _IN), lambda i: (2 * i + 1, 0)),
            pl.BlockSpec((_IN, _H2), lambda i: (0, 0)),
            pl.BlockSpec((1, _H2), lambda i: (0, 0)),
            pl.BlockSpec((_H2, _OUT), lambda i: (0, 0)),
            pl.BlockSpec((1, _OUT), lambda i: (0, 0)),
            pl.BlockSpec((_H2, _OUT), lambda i: (0, 0)),
            pl.BlockSpec((1, _OUT), lambda i: (0, 0)),
        ],
        out_specs=(pl.BlockSpec((TB, _OUT), lambda i: (i, 0)),
                   pl.BlockSpec((TB, _OUT), lambda i: (i, 0))),
        compiler_params=pltpu.CompilerParams(
            dimension_semantics=("parallel",)),
        cost_estimate=pl.CostEstimate(
            flops=flops, transcendentals=Bp * _OUT,
            bytes_accessed=bytes_accessed),
    )(x, x, w1u, b1u, w2m, b2m, w2a, b2a)

    if Bp != B:
        aug = aug[:B]
        ml = ml[:B]
    return aug, ml


def kernel(x, w1, b1, w2, b2):
    return _forward(x, w1, b1, w2, b2)


# speedup vs baseline: 2.6903x; 2.6903x over previous
import functools

import jax
import jax.numpy as jnp
from jax.experimental import pallas as pl
from jax.experimental.pallas import tpu as pltpu
from jax.experimental.layout import Format, Layout

_IN = 16
_H2 = 64
_OUT = 16


def _fused_kernel(x_ref, w1t_ref, b1c_ref, w2mt_ref, b2mc_ref,
                  w2at_ref, b2ac_ref, augt_ref, mlt_ref):
    x = x_ref[...]
    ht = jax.lax.dot_general(
        w1t_ref[...], x, (((1,), (1,)), ((), ())),
        preferred_element_type=jnp.float32)
    ht = jnp.maximum(ht + b1c_ref[...], 0.0)

    mlt_ref[...] = (jnp.dot(w2mt_ref[...], ht,
                            preferred_element_type=jnp.float32)
                    + b2mc_ref[...])

    at = (jnp.dot(w2at_ref[...], ht, preferred_element_type=jnp.float32)
          + b2ac_ref[...])
    m = jnp.max(at, axis=0, keepdims=True)
    s = at - m
    lse = jnp.log(jnp.sum(jnp.exp(s), axis=0, keepdims=True))
    augt_ref[...] = s - lse


def _forward(x, w1, b1, w2, b2):
    x = x.astype(jnp.float32)
    B = x.shape[0]

    w1t = jax.lax.slice(w1, (0, 0), (_IN, _H2)).T
    b1c = jax.lax.slice(b1, (0, 0), (1, _H2)).T
    w2mt = jax.lax.slice(w2, (0, 0), (_H2, _OUT)).T
    b2mc = jax.lax.slice(b2, (0, 0), (1, _OUT)).T
    w2at = jax.lax.slice(w2, (0, _OUT), (_H2, 2 * _OUT)).T
    b2ac = jax.lax.slice(b2, (0, _OUT), (1, 2 * _OUT)).T

    TB = 16384
    num_tiles = pl.cdiv(B, TB)
    Bp = num_tiles * TB
    if Bp != B:
        x = jnp.pad(x, ((0, Bp - B), (0, 0)))

    flops = 2 * Bp * (_IN * _H2 + _H2 * 2 * _OUT)
    bytes_accessed = 4 * (Bp * (_IN + 2 * _OUT)
                          + _IN * _H2 + _H2 * 2 * _OUT + _H2 + 2 * _OUT)

    augt, mlt = pl.pallas_call(
        _fused_kernel,
        out_shape=(jax.ShapeDtypeStruct((_OUT, Bp), jnp.float32),
                   jax.ShapeDtypeStruct((_OUT, Bp), jnp.float32)),
        grid=(num_tiles,),
        in_specs=[
            pl.BlockSpec((TB, _IN), lambda i: (i, 0)),
            pl.BlockSpec((_H2, _IN), lambda i: (0, 0)),
            pl.BlockSpec((_H2, 1), lambda i: (0, 0)),
            pl.BlockSpec((_OUT, _H2), lambda i: (0, 0)),
            pl.BlockSpec((_OUT, 1), lambda i: (0, 0)),
            pl.BlockSpec((_OUT, _H2), lambda i: (0, 0)),
            pl.BlockSpec((_OUT, 1), lambda i: (0, 0)),
        ],
        out_specs=(pl.BlockSpec((_OUT, TB), lambda i: (0, i)),
                   pl.BlockSpec((_OUT, TB), lambda i: (0, i))),
        compiler_params=pltpu.CompilerParams(
            dimension_semantics=("parallel",)),
        cost_estimate=pl.CostEstimate(
            flops=flops, transcendentals=Bp * _OUT,
            bytes_accessed=bytes_accessed),
    )(x, w1t, b1c, w2mt, b2mc, w2at, b2ac)

    aug = augt[:, :B].T
    ml = mlt[:, :B].T
    return aug, ml


@functools.cache
def _jitted(sharding):
    fmt = Format(Layout(major_to_minor=(1, 0)), sharding)
    return jax.jit(_forward, out_shardings=(fmt, fmt))


def kernel(x, w1, b1, w2, b2):
    sh = jax.sharding.SingleDeviceSharding(jax.devices()[0])
    return _jitted(sh)(x, w1, b1, w2, b2)


# speedup vs baseline: 2.7219x; 1.0118x over previous
import functools

import jax
import jax.numpy as jnp
from jax.experimental import pallas as pl
from jax.experimental.pallas import tpu as pltpu
from jax.experimental.layout import Format, Layout

_IN = 16
_H2 = 64
_OUT = 16


def _fused_kernel(x_ref, w1t_ref, b1c_ref, w2mt_ref, b2mc_ref,
                  w2at_ref, b2ac_ref, augt_ref, mlt_ref):
    x = x_ref[...]
    ht = jax.lax.dot_general(
        w1t_ref[...], x, (((1,), (1,)), ((), ())),
        preferred_element_type=jnp.float32)
    ht = jnp.maximum(ht + b1c_ref[...], 0.0)

    mlt_ref[...] = (jnp.dot(w2mt_ref[...], ht,
                            preferred_element_type=jnp.float32)
                    + b2mc_ref[...])

    at = (jnp.dot(w2at_ref[...], ht, preferred_element_type=jnp.float32)
          + b2ac_ref[...])
    m = jnp.max(at, axis=0, keepdims=True)
    s = at - m
    lse = jnp.log(jnp.sum(jnp.exp(s), axis=0, keepdims=True))
    augt_ref[...] = s - lse


def _forward(x, w1, b1, w2, b2):
    x = x.astype(jnp.float32)
    B = x.shape[0]

    w1t = jax.lax.slice(w1, (0, 0), (_IN, _H2)).T
    b1c = jax.lax.slice(b1, (0, 0), (1, _H2)).T
    w2mt = jax.lax.slice(w2, (0, 0), (_H2, _OUT)).T
    b2mc = jax.lax.slice(b2, (0, 0), (1, _OUT)).T
    w2at = jax.lax.slice(w2, (0, _OUT), (_H2, 2 * _OUT)).T
    b2ac = jax.lax.slice(b2, (0, _OUT), (1, 2 * _OUT)).T

    TB = 32768
    num_tiles = pl.cdiv(B, TB)
    Bp = num_tiles * TB
    if Bp != B:
        x = jnp.pad(x, ((0, Bp - B), (0, 0)))

    flops = 2 * Bp * (_IN * _H2 + _H2 * 2 * _OUT)
    bytes_accessed = 4 * (Bp * (_IN + 2 * _OUT)
                          + _IN * _H2 + _H2 * 2 * _OUT + _H2 + 2 * _OUT)

    augt, mlt = pl.pallas_call(
        _fused_kernel,
        out_shape=(jax.ShapeDtypeStruct((_OUT, Bp), jnp.float32),
                   jax.ShapeDtypeStruct((_OUT, Bp), jnp.float32)),
        grid=(num_tiles,),
        in_specs=[
            pl.BlockSpec((TB, _IN), lambda i: (i, 0)),
            pl.BlockSpec((_H2, _IN), lambda i: (0, 0)),
            pl.BlockSpec((_H2, 1), lambda i: (0, 0)),
            pl.BlockSpec((_OUT, _H2), lambda i: (0, 0)),
            pl.BlockSpec((_OUT, 1), lambda i: (0, 0)),
            pl.BlockSpec((_OUT, _H2), lambda i: (0, 0)),
            pl.BlockSpec((_OUT, 1), lambda i: (0, 0)),
        ],
        out_specs=(pl.BlockSpec((_OUT, TB), lambda i: (0, i)),
                   pl.BlockSpec((_OUT, TB), lambda i: (0, i))),
        compiler_params=pltpu.CompilerParams(
            dimension_semantics=("parallel",)),
        cost_estimate=pl.CostEstimate(
            flops=flops, transcendentals=Bp * _OUT,
            bytes_accessed=bytes_accessed),
    )(x, w1t, b1c, w2mt, b2mc, w2at, b2ac)

    aug = augt[:, :B].T
    ml = mlt[:, :B].T
    return aug, ml


@functools.cache
def _jitted(sharding):
    fmt = Format(Layout(major_to_minor=(1, 0)), sharding)
    return jax.jit(_forward, out_shardings=(fmt, fmt))


def kernel(x, w1, b1, w2, b2):
    sh = jax.sharding.SingleDeviceSharding(jax.devices()[0])
    return _jitted(sh)(x, w1, b1, w2, b2)
